# baseline (speedup 1.0000x reference)
import jax
import jax.numpy as jnp
from jax.experimental import pallas as pl
from jax.experimental.pallas import tpu as pltpu

_B, _T, _N = 8, 512, 64
_D_IN, _D = 15, 128
_M = _B * _T * _N
_EPS = 1e-5

_R_STATS = 4096
_R_MAIN = 2048


def _stats_body(x_ref, m_ref, w1_ref, g_ref, bt_ref, w1p_ref, b1p_ref,
                acc_mat, acc_s, acc_n):
    i = pl.program_id(0)
    x = x_ref[...]
    m = m_ref[...]
    xm = x * m
    p_mat = jax.lax.dot_general(xm, x, (((0,), (0,)), ((), ())),
                                preferred_element_type=jnp.float32)
    p_s = jnp.sum(xm, axis=0, keepdims=True)
    p_n = jnp.sum(m, axis=(0, 1), keepdims=True)

    @pl.when(i == 0)
    def _init():
        acc_mat[...] = jnp.zeros_like(acc_mat)
        acc_s[...] = jnp.zeros_like(acc_s)
        acc_n[...] = jnp.zeros_like(acc_n)

    acc_mat[...] += p_mat
    acc_s[...] += p_s
    acc_n[...] += p_n

    @pl.when(i == pl.num_programs(0) - 1)
    def _finalize():
        w1 = w1_ref[...]
        n = acc_n[0, 0]
        mu = jnp.dot(acc_s[...], w1, preferred_element_type=jnp.float32) / n
        e2 = jnp.sum(jnp.dot(acc_mat[...], w1,
                             preferred_element_type=jnp.float32) * w1,
                     axis=0, keepdims=True) / n
        var = e2 - mu * mu
        a = g_ref[...] * jax.lax.rsqrt(var + _EPS)
        w1p_ref[...] = w1 * a
        b1p_ref[...] = bt_ref[...] - mu * a


def _main_body(x_ref, m_ref, w1p_ref, b1p_ref, w2_ref, b2_ref, o_ref):
    x = x_ref[...]
    h = jnp.dot(x, w1p_ref[...], preferred_element_type=jnp.float32)
    h = jnp.maximum(h + b1p_ref[...], 0.0)
    o = jnp.dot(h, w2_ref[...], preferred_element_type=jnp.float32)
    o_ref[...] = (o + b2_ref[...]) * m_ref[...]


def kernel(diff, W1, gamma, beta, W2, b2, mask, batch_dict):
    x = diff.reshape(_M, _D_IN)
    mf = mask.reshape(_M, 1).astype(jnp.float32)
    g2 = gamma.reshape(1, _D)
    bt2 = beta.reshape(1, _D)
    b22 = b2.reshape(1, _D)

    w1p, b1p = pl.pallas_call(
        _stats_body,
        grid=(_M // _R_STATS,),
        in_specs=[
            pl.BlockSpec((_R_STATS, _D_IN), lambda i: (i, 0)),
            pl.BlockSpec((_R_STATS, 1), lambda i: (i, 0)),
            pl.BlockSpec((_D_IN, _D), lambda i: (0, 0)),
            pl.BlockSpec((1, _D), lambda i: (0, 0)),
            pl.BlockSpec((1, _D), lambda i: (0, 0)),
        ],
        out_specs=[
            pl.BlockSpec((_D_IN, _D), lambda i: (0, 0)),
            pl.BlockSpec((1, _D), lambda i: (0, 0)),
        ],
        out_shape=[
            jax.ShapeDtypeStruct((_D_IN, _D), jnp.float32),
            jax.ShapeDtypeStruct((1, _D), jnp.float32),
        ],
        scratch_shapes=[
            pltpu.VMEM((_D_IN, _D_IN), jnp.float32),
            pltpu.VMEM((1, _D_IN), jnp.float32),
            pltpu.VMEM((1, 1), jnp.float32),
        ],
        compiler_params=pltpu.CompilerParams(
            dimension_semantics=("arbitrary",)),
    )(x, mf, W1, g2, bt2)

    out = pl.pallas_call(
        _main_body,
        grid=(_M // _R_MAIN,),
        in_specs=[
            pl.BlockSpec((_R_MAIN, _D_IN), lambda i: (i, 0)),
            pl.BlockSpec((_R_MAIN, 1), lambda i: (i, 0)),
            pl.BlockSpec((_D_IN, _D), lambda i: (0, 0)),
            pl.BlockSpec((1, _D), lambda i: (0, 0)),
            pl.BlockSpec((_D, _D), lambda i: (0, 0)),
            pl.BlockSpec((1, _D), lambda i: (0, 0)),
        ],
        out_specs=pl.BlockSpec((_R_MAIN, _D), lambda i: (i, 0)),
        out_shape=jax.ShapeDtypeStruct((_M, _D), jnp.float32),
        compiler_params=pltpu.CompilerParams(
            dimension_semantics=("arbitrary",)),
    )(x, mf, w1p, b1p, W2, b22)

    return out.reshape(_B, _T, _N, _D), mask
